# baseline (device time: 98301 ns/iter reference)
import jax
import jax.numpy as jnp
from jax import lax
from jax.experimental import pallas as pl
from jax.experimental.pallas import tpu as pltpu

N_DEV = 8
B = 2
SQ = 128
SKV = 128
HQ_LOCAL = 4
DH = 64
DM = 512
DQ = HQ_LOCAL * DH


def kernel(x, Wq, K_ext, V_ext, Wo):
    def body(x_ref, wq_ref, k_ref, v_ref, wo_ref, out_ref,
             wq_all, wo_all, wq_send, wq_recv, wo_send, wo_recv):
        my = lax.axis_index("i")
        left = lax.rem(my - 1 + N_DEV, N_DEV)
        right = lax.rem(my + 1, N_DEV)

        barrier_sem = pltpu.get_barrier_semaphore()
        for nbr in (left, right):
            pl.semaphore_signal(barrier_sem, inc=1, device_id=(nbr,),
                                device_id_type=pl.DeviceIdType.MESH)
        pl.semaphore_wait(barrier_sem, 2)

        wq_all[pl.ds(my, 1)] = wq_ref[...].astype(jnp.bfloat16)[None]
        wo_all[pl.ds(my, 1)] = wo_ref[...].astype(jnp.bfloat16)[None]

        for h in range(N_DEV - 1):
            s = lax.rem(my - h + N_DEV, N_DEV)
            wq_rdma = pltpu.make_async_remote_copy(
                src_ref=wq_all.at[s], dst_ref=wq_all.at[s],
                send_sem=wq_send.at[h], recv_sem=wq_recv.at[h],
                device_id=(right,), device_id_type=pl.DeviceIdType.MESH,
            )
            wo_rdma = pltpu.make_async_remote_copy(
                src_ref=wo_all.at[s], dst_ref=wo_all.at[s],
                send_sem=wo_send.at[h], recv_sem=wo_recv.at[h],
                device_id=(right,), device_id_type=pl.DeviceIdType.MESH,
            )
            wq_rdma.start()
            wo_rdma.start()
            wq_rdma.wait()
            wo_rdma.wait()

        iq = lax.broadcasted_iota(jnp.int32, (SQ, SKV), 0)
        ik = lax.broadcasted_iota(jnp.int32, (SQ, SKV), 1)
        qb = my * 2 + iq // 64
        kb = ik // 64
        mask = (qb == kb) | (lax.rem(kb, 4) == lax.rem(qb, 4))
        row_keep = jnp.any(mask, axis=1, keepdims=True)

        x_bf = x_ref[...].astype(jnp.bfloat16)

        for b in range(B):
            acc = jnp.zeros((SQ, DM), jnp.float32)
            xb = x_bf[b]
            for j in range(N_DEV):
                qj = jnp.dot(xb, wq_all[j],
                             preferred_element_type=jnp.float32)
                qj = qj.astype(jnp.bfloat16)
                for hh in range(HQ_LOCAL):
                    head = j * HQ_LOCAL + hh
                    q = qj[:, hh * DH:(hh + 1) * DH]
                    k = k_ref[b, :, head, :].astype(jnp.bfloat16)
                    v = v_ref[b, :, head, :].astype(jnp.bfloat16)
                    scores = jnp.dot(q, k.T,
                                     preferred_element_type=jnp.float32)
                    scores = scores * 0.125
                    scores = jnp.where(mask, scores, -1e9)
                    m = jnp.max(scores, axis=-1, keepdims=True)
                    w = jnp.exp(scores - m)
                    w = w / jnp.sum(w, axis=-1, keepdims=True)
                    w = jnp.where(row_keep, w, 0.0)
                    ctx = jnp.dot(w.astype(jnp.bfloat16), v,
                                  preferred_element_type=jnp.float32)
                    acc = acc + jnp.dot(
                        ctx.astype(jnp.bfloat16),
                        wo_all[j][hh * DH:(hh + 1) * DH, :],
                        preferred_element_type=jnp.float32)
            out_ref[b] = acc

    return pl.pallas_call(
        body,
        out_shape=jax.ShapeDtypeStruct((B, SQ, DM), jnp.float32),
        in_specs=[pl.BlockSpec(memory_space=pltpu.VMEM)] * 5,
        out_specs=pl.BlockSpec(memory_space=pltpu.VMEM),
        scratch_shapes=[
            pltpu.VMEM((N_DEV, DM, DQ), jnp.bfloat16),
            pltpu.VMEM((N_DEV, DQ, DM), jnp.bfloat16),
            pltpu.SemaphoreType.DMA((N_DEV - 1,)),
            pltpu.SemaphoreType.DMA((N_DEV - 1,)),
            pltpu.SemaphoreType.DMA((N_DEV - 1,)),
            pltpu.SemaphoreType.DMA((N_DEV - 1,)),
        ],
        compiler_params=pltpu.CompilerParams(collective_id=0),
    )(x, Wq, K_ext, V_ext, Wo)


# device time: 70625 ns/iter; 1.3919x vs baseline; 1.3919x over previous
import jax
import jax.numpy as jnp
from jax import lax
from jax.experimental import pallas as pl
from jax.experimental.pallas import tpu as pltpu

N_DEV = 8
B = 2
SQ = 128
SKV = 128
HQ_LOCAL = 4
DH = 64
DM = 512
DQ = HQ_LOCAL * DH


def kernel(x, Wq, K_ext, V_ext, Wo):
    def body(x_ref, wq_ref, k_ref, v_ref, wo_ref, out_ref,
             wq_all, wo_all, wq_send, wq_recv, wo_send, wo_recv):
        my = lax.axis_index("i")

        barrier_sem = pltpu.get_barrier_semaphore()
        for k in range(1, N_DEV):
            peer = lax.rem(my + k, N_DEV)
            pl.semaphore_signal(barrier_sem, inc=1, device_id=(peer,),
                                device_id_type=pl.DeviceIdType.MESH)
        pl.semaphore_wait(barrier_sem, N_DEV - 1)

        wq_all[pl.ds(my, 1)] = wq_ref[...].astype(jnp.bfloat16)[None]
        wo_all[pl.ds(my, 1)] = wo_ref[...].astype(jnp.bfloat16)[None]

        rdmas = []
        for k in range(1, N_DEV):
            peer = lax.rem(my + k, N_DEV)
            wq_rdma = pltpu.make_async_remote_copy(
                src_ref=wq_all.at[my], dst_ref=wq_all.at[my],
                send_sem=wq_send.at[k - 1], recv_sem=wq_recv.at[my],
                device_id=(peer,), device_id_type=pl.DeviceIdType.MESH,
            )
            wo_rdma = pltpu.make_async_remote_copy(
                src_ref=wo_all.at[my], dst_ref=wo_all.at[my],
                send_sem=wo_send.at[k - 1], recv_sem=wo_recv.at[my],
                device_id=(peer,), device_id_type=pl.DeviceIdType.MESH,
            )
            wq_rdma.start()
            wo_rdma.start()
            rdmas.append((wq_rdma, wo_rdma))

        iq = lax.broadcasted_iota(jnp.int32, (SQ, SKV), 0)
        ik = lax.broadcasted_iota(jnp.int32, (SQ, SKV), 1)
        qb = my * 2 + iq // 64
        kb = ik // 64
        mask = (qb == kb) | (lax.rem(kb, 4) == lax.rem(qb, 4))
        row_keep = jnp.any(mask, axis=1, keepdims=True)

        x_bf = x_ref[...].astype(jnp.bfloat16)

        accs = [jnp.zeros((SQ, DM), jnp.float32) for _ in range(B)]
        for j in range(N_DEV):
            @pl.when(j != my)
            def _():
                wq_wait = pltpu.make_async_remote_copy(
                    src_ref=wq_all.at[j], dst_ref=wq_all.at[j],
                    send_sem=wq_send.at[0], recv_sem=wq_recv.at[j],
                    device_id=(my,), device_id_type=pl.DeviceIdType.MESH,
                )
                wo_wait = pltpu.make_async_remote_copy(
                    src_ref=wo_all.at[j], dst_ref=wo_all.at[j],
                    send_sem=wo_send.at[0], recv_sem=wo_recv.at[j],
                    device_id=(my,), device_id_type=pl.DeviceIdType.MESH,
                )
                wq_wait.wait_recv()
                wo_wait.wait_recv()

            for b in range(B):
                qj = jnp.dot(x_bf[b], wq_all[j],
                             preferred_element_type=jnp.float32)
                qj = qj.astype(jnp.bfloat16)
                ctx_heads = []
                for hh in range(HQ_LOCAL):
                    head = j * HQ_LOCAL + hh
                    q = qj[:, hh * DH:(hh + 1) * DH]
                    kk = k_ref[b, :, head, :].astype(jnp.bfloat16)
                    vv = v_ref[b, :, head, :].astype(jnp.bfloat16)
                    scores = jnp.dot(q, kk.T,
                                     preferred_element_type=jnp.float32)
                    scores = scores * 0.125
                    scores = jnp.where(mask, scores, -1e9)
                    m = jnp.max(scores, axis=-1, keepdims=True)
                    w = jnp.exp(scores - m)
                    w = w / jnp.sum(w, axis=-1, keepdims=True)
                    w = jnp.where(row_keep, w, 0.0)
                    ctx_heads.append(jnp.dot(w.astype(jnp.bfloat16), vv,
                                             preferred_element_type=jnp.float32))
                ctx = jnp.concatenate(ctx_heads, axis=1)
                accs[b] = accs[b] + jnp.dot(
                    ctx.astype(jnp.bfloat16), wo_all[j],
                    preferred_element_type=jnp.float32)

        for b in range(B):
            out_ref[b] = accs[b]

        for wq_rdma, wo_rdma in rdmas:
            wq_rdma.wait_send()
            wo_rdma.wait_send()

    return pl.pallas_call(
        body,
        out_shape=jax.ShapeDtypeStruct((B, SQ, DM), jnp.float32),
        in_specs=[pl.BlockSpec(memory_space=pltpu.VMEM)] * 5,
        out_specs=pl.BlockSpec(memory_space=pltpu.VMEM),
        scratch_shapes=[
            pltpu.VMEM((N_DEV, DM, DQ), jnp.bfloat16),
            pltpu.VMEM((N_DEV, DQ, DM), jnp.bfloat16),
            pltpu.SemaphoreType.DMA((N_DEV - 1,)),
            pltpu.SemaphoreType.DMA((N_DEV,)),
            pltpu.SemaphoreType.DMA((N_DEV - 1,)),
            pltpu.SemaphoreType.DMA((N_DEV,)),
        ],
        compiler_params=pltpu.CompilerParams(collective_id=0),
    )(x, Wq, K_ext, V_ext, Wo)
